# baseline (speedup 1.0000x reference)
import jax
import jax.numpy as jnp
from jax import lax
from jax.experimental import pallas as pl
from jax.experimental.pallas import tpu as pltpu
from jax.experimental.pallas import tpu_sc as plsc

_NC, _NS = 2, 16
_NW = _NC * _NS
_B, _S, _D = 4096, 200, 64
_DP = 128
_ROWS_PER_W = _B // _NW
_PLEN = _S * _DP
_TABLE_ROWS = 101
_TLEN = _TABLE_ROWS * _D
_MPAD = 208
_NB = 4
_NCH = _ROWS_PER_W


def _sc_body(m_hbm, t_hbm, pe_hbm, tabv, descv,
             mbuf0, mbuf1, mbuf2, mbuf3, obuf0, obuf1, obuf2, obuf3,
             min_sem, out_sem):
    mbufs = [mbuf0, mbuf1, mbuf2, mbuf3]
    obufs = [obuf0, obuf1, obuf2, obuf3]
    wid = lax.axis_index("s") * _NC + lax.axis_index("c")
    row0 = wid * _ROWS_PER_W

    pltpu.sync_copy(t_hbm, tabv)

    def _rev(j, carry):
        for g in range(4):
            descv[pl.ds(j * _D + g * 16, 16)] = tabv[pl.ds((100 - j) * _D + g * 16, 16)]
        return carry

    lax.fori_loop(0, 100, _rev, 0)

    zf = jnp.zeros((16,), jnp.float32)

    def _zero(k, carry):
        for ob in obufs:
            ob[pl.ds(k * 16, 16)] = zf
        return carry

    lax.fori_loop(0, _PLEN // 16, _zero, 0)

    def _start_mask(c, b):
        pltpu.async_copy(m_hbm.at[pl.ds((row0 + c) * _MPAD, _MPAD)], mbufs[b],
                         min_sem.at[b])

    def _wait_out(b):
        pltpu.make_async_copy(obufs[b], pe_hbm.at[pl.ds(0, _PLEN)],
                              out_sem.at[b]).wait()

    def _chunk(c, b, first_round):
        pltpu.make_async_copy(m_hbm.at[pl.ds(0, _MPAD)], mbufs[b],
                              min_sem.at[b]).wait()
        acc = mbufs[b][pl.ds(0, 16)]
        for cc in range(1, _MPAD // 16):
            acc = acc + mbufs[b][pl.ds(cc * 16, 16)]
        vl = acc[0]
        for l in range(1, 16):
            vl = vl + acc[l]
        h2 = vl // 2
        h1 = vl - h2

        if not first_round:
            _wait_out(b)

        ob = obufs[b]

        def _asc(p, c2):
            for g in range(4):
                ob[pl.ds(p * _DP + g * 16, 16)] = tabv[pl.ds(_D + p * _D + g * 16, 16)]
            return c2

        lax.fori_loop(0, h1, _asc, 0)

        doff = (100 - h2) * _D

        def _desc(p, c2):
            for g in range(4):
                ob[pl.ds((h1 + p) * _DP + g * 16, 16)] = descv[pl.ds(doff + p * _D + g * 16, 16)]
            return c2

        lax.fori_loop(0, h2, _desc, 0)

        def _tail(p, c2):
            for g in range(4):
                ob[pl.ds(p * _DP + g * 16, 16)] = zf
            return c2

        lax.fori_loop(vl, _S, _tail, 0)

        pltpu.async_copy(ob, pe_hbm.at[pl.ds((row0 + c) * _PLEN, _PLEN)],
                         out_sem.at[b])

    _start_mask(0, 0)
    _start_mask(1, 1)

    @pl.loop(0, _NCH, step=_NB)
    def _outer(g):
        for b in range(_NB):
            c = g + b

            @pl.when(c + 2 < _NCH)
            def _pf():
                _start_mask(c + 2, (b + 2) % _NB)

            @pl.when(g > 0)
            def _cn():
                _chunk(c, b, False)

            @pl.when(g == 0)
            def _c0():
                _chunk(c, b, True)

    for b in range(_NB):
        _wait_out(b)


def _tc_add_body(x_ref, pe_ref, o_ref):
    o_ref[...] = x_ref[...] + pe_ref[:, :, :_D]


def kernel(x, mask, position_embedding):
    b, s, d = x.shape
    mi = jnp.pad(mask.astype(jnp.int32), ((0, 0), (0, _MPAD - s))).reshape(-1)
    tf = position_embedding.reshape(-1)
    mesh = plsc.VectorSubcoreMesh(
        core_axis_name="c", subcore_axis_name="s", num_cores=_NC, num_subcores=_NS
    )
    pe_flat = pl.kernel(
        _sc_body,
        out_type=jax.ShapeDtypeStruct((b * s * _DP,), jnp.float32),
        mesh=mesh,
        scratch_types=[
            pltpu.VMEM((_TLEN,), jnp.float32),
            pltpu.VMEM((100 * _D,), jnp.float32),
            pltpu.VMEM((_MPAD,), jnp.int32),
            pltpu.VMEM((_MPAD,), jnp.int32),
            pltpu.VMEM((_MPAD,), jnp.int32),
            pltpu.VMEM((_MPAD,), jnp.int32),
            pltpu.VMEM((_PLEN,), jnp.float32),
            pltpu.VMEM((_PLEN,), jnp.float32),
            pltpu.VMEM((_PLEN,), jnp.float32),
            pltpu.VMEM((_PLEN,), jnp.float32),
            pltpu.SemaphoreType.DMA((_NB,)),
            pltpu.SemaphoreType.DMA((_NB,)),
        ],
    )(mi, tf)
    pe = pe_flat.reshape(b, s, _DP)

    bb = 8
    out = pl.pallas_call(
        _tc_add_body,
        out_shape=jax.ShapeDtypeStruct((b, s, d), x.dtype),
        grid=(b // bb,),
        in_specs=[
            pl.BlockSpec((bb, s, d), lambda i: (i, 0, 0)),
            pl.BlockSpec((bb, s, _DP), lambda i: (i, 0, 0)),
        ],
        out_specs=pl.BlockSpec((bb, s, d), lambda i: (i, 0, 0)),
    )(x, pe)
    return out
